# Initial kernel scaffold; baseline (speedup 1.0000x reference)
import jax
import jax.numpy as jnp
from jax.experimental import pallas as pl


def kernel(robot_x, human_x, edge_index, Wr1, br1, Wr2, br2, Wh1, bh1, Wh2, bh2, W0, W1, W2, bias):
    raise NotImplementedError("write your pallas kernel here")


# speedup vs baseline: 59.4176x; 59.4176x over previous
import functools

import jax
import jax.numpy as jnp
from jax import lax
from jax.experimental import pallas as pl
from jax.experimental.pallas import tpu as pltpu
from jax.experimental.pallas import tpu_sc as plsc

_L = 128
_SLAB = 16
_NW = 32
_BLK = 2048


def _wid():
    return lax.axis_index("s") * 2 + lax.axis_index("c")


def _row_range(rows):
    wid = _wid()
    base, ext = rows // _NW, rows % _NW
    nrows = base + jnp.where(wid < ext, 1, 0).astype(jnp.int32)
    row0 = wid * base + jnp.minimum(wid, ext)
    return row0, nrows


def _zero_spmem(zbuf, sps, seg):
    sid = lax.axis_index("s")
    for i in range(seg // 16):
        zbuf[pl.ds(i * 16, 16)] = jnp.zeros((16,), jnp.float32)
    for sp in sps:
        pltpu.sync_copy(zbuf, sp.at[pl.ds(sid * seg, seg)])


def _copy_out(zbuf, sp, out, seg):
    cid = lax.axis_index("c")
    sid = lax.axis_index("s")
    sl = pl.ds(sid * seg, seg)
    pltpu.sync_copy(sp.at[sl], zbuf)
    pltpu.sync_copy(zbuf, out.at[cid, sl])


def _histo_kernel(rows, npad):
    seg = npad // 16
    mesh = plsc.VectorSubcoreMesh(core_axis_name="c", subcore_axis_name="s")

    @functools.partial(
        pl.kernel, mesh=mesh,
        compiler_params=pltpu.CompilerParams(use_tc_tiling_on_sc=False, needs_layout_passes=False),
        out_type=[jax.ShapeDtypeStruct((2, npad), jnp.float32),
                  jax.ShapeDtypeStruct((2, npad), jnp.float32)],
        scratch_types=[
            pltpu.VMEM((_SLAB, _L), jnp.int32),
            pltpu.VMEM((_SLAB, _L), jnp.int32),
            pltpu.VMEM((_L,), jnp.float32),
            pltpu.VMEM((_L,), jnp.float32),
            pltpu.VMEM((seg,), jnp.float32),
            pltpu.VMEM_SHARED((npad,), jnp.float32),
            pltpu.VMEM_SHARED((npad,), jnp.float32),
        ],
    )
    def k1(src_hbm, dst_hbm, deg_out, cun_out,
           srcb, dstb, valb, ones, zbuf, deg_sp, cun_sp):
        _zero_spmem(zbuf, (deg_sp, cun_sp), seg)
        for i in range(_L // 16):
            ones[pl.ds(i * 16, 16)] = jnp.ones((16,), jnp.float32)
        plsc.subcore_barrier()

        row0, nrows = _row_range(rows)

        def do_row(j):
            idx = srcb.at[j]
            pltpu.sync_copy(ones, deg_sp.at[idx], add=True)
            m = dstb[j, pl.ds(0, 16)] == 0
            for k in range(1, _L // 16):
                m = m | (dstb[j, pl.ds(k * 16, 16)] == 0)
            has0 = lax.reduce_or(m, axes=(0,))

            @pl.when(has0)
            def _():
                for k in range(_L // 16):
                    valb[pl.ds(k * 16, 16)] = jnp.where(
                        dstb[j, pl.ds(k * 16, 16)] == 0, 1.0, 0.0)
                pltpu.sync_copy(valb, cun_sp.at[idx], add=True)

        nslab = nrows // _SLAB

        def slab_body(s, c):
            r0 = row0 + s * _SLAB
            pltpu.sync_copy(src_hbm.at[pl.ds(r0, _SLAB), :], srcb)
            pltpu.sync_copy(dst_hbm.at[pl.ds(r0, _SLAB), :], dstb)
            for j in range(_SLAB):
                do_row(j)
            return c

        lax.fori_loop(0, nslab, slab_body, 0)

        def tail_body(t, c):
            r = row0 + nslab * _SLAB + t
            pltpu.sync_copy(src_hbm.at[r], srcb.at[0])
            pltpu.sync_copy(dst_hbm.at[r], dstb.at[0])
            do_row(0)
            return c

        lax.fori_loop(0, nrows - nslab * _SLAB, tail_body, 0)

        plsc.subcore_barrier()
        _copy_out(zbuf, deg_sp, deg_out, seg)
        _copy_out(zbuf, cun_sp, cun_out, seg)

    return k1


def _prop_kernel(rows, npad):
    seg = npad // 16
    mesh = plsc.VectorSubcoreMesh(core_axis_name="c", subcore_axis_name="s")

    @functools.partial(
        pl.kernel, mesh=mesh,
        compiler_params=pltpu.CompilerParams(use_tc_tiling_on_sc=False, needs_layout_passes=False),
        out_type=jax.ShapeDtypeStruct((2, npad), jnp.float32),
        scratch_types=[
            pltpu.VMEM((_SLAB, _L), jnp.int32),
            pltpu.VMEM((_SLAB, _L), jnp.int32),
            pltpu.VMEM((_L,), jnp.float32),
            pltpu.VMEM((seg,), jnp.float32),
            pltpu.VMEM_SHARED((npad,), jnp.float32),
        ],
    )
    def k3(src_hbm, dst_hbm, cd_hbm, hp_out, srcb, dstb, gbuf, zbuf, hp_sp):
        _zero_spmem(zbuf, (hp_sp,), seg)
        plsc.subcore_barrier()

        row0, nrows = _row_range(rows)

        def do_row(j):
            pltpu.sync_copy(cd_hbm.at[dstb.at[j]], gbuf)
            pltpu.sync_copy(gbuf, hp_sp.at[srcb.at[j]], add=True)

        nslab = nrows // _SLAB

        def slab_body(s, c):
            r0 = row0 + s * _SLAB
            pltpu.sync_copy(src_hbm.at[pl.ds(r0, _SLAB), :], srcb)
            pltpu.sync_copy(dst_hbm.at[pl.ds(r0, _SLAB), :], dstb)
            for j in range(_SLAB):
                do_row(j)
            return c

        lax.fori_loop(0, nslab, slab_body, 0)

        def tail_body(t, c):
            r = row0 + nslab * _SLAB + t
            pltpu.sync_copy(src_hbm.at[r], srcb.at[0])
            pltpu.sync_copy(dst_hbm.at[r], dstb.at[0])
            do_row(0)
            return c

        lax.fori_loop(0, nrows - nslab * _SLAB, tail_body, 0)

        plsc.subcore_barrier()
        _copy_out(zbuf, hp_sp, hp_out, seg)

    return k3


def _tables_body(dp, cp, hx, rx, Wh1, bh1, Wh2, bh2, Wr1, br1, Wr2, br2,
                 X_ref, t1_ref, cd_ref, dis_ref):
    deg = dp[0:1, :] + dp[1:2, :]
    pos = deg > 0.0
    dis = jnp.where(pos, lax.rsqrt(jnp.maximum(deg, 1e-12)), 0.0)
    cun = cp[0:1, :] + cp[1:2, :]
    t1_ref[...] = cun * dis
    cd_ref[...] = jnp.where(pos, cun / jnp.maximum(deg, 1e-12), 0.0)
    dis_ref[...] = dis
    h1 = jax.nn.relu(
        jnp.dot(hx[...], Wh1[...], preferred_element_type=jnp.float32)
        + bh1[...])
    X_ref[...] = jax.nn.relu(
        jnp.dot(h1, Wh2[...], preferred_element_type=jnp.float32) + bh2[...])

    @pl.when(pl.program_id(0) == 0)
    def _():
        r1 = jax.nn.relu(
            jnp.dot(rx[...], Wr1[...], preferred_element_type=jnp.float32)
            + br1[...])
        r = jax.nn.relu(
            jnp.dot(r1, Wr2[...], preferred_element_type=jnp.float32)
            + br2[...])
        X_ref[0:1, :] = r


def _final_body(X, t1, dis, hp, rx, Wr1, br1, Wr2, br2, W0, W1, W2, bias,
                out_ref, acc, dis0):
    pid = pl.program_id(0)
    last = pl.num_programs(0) - 1

    @pl.when(pid == 0)
    def _():
        acc[...] = jnp.zeros_like(acc)
        dis0[0] = dis[0, 0]

    g = dis[...] * (hp[0:1, :] + hp[1:2, :])
    v1 = jnp.dot(t1[...], X[...], preferred_element_type=jnp.float32)
    v2 = jnp.dot(g, X[...], preferred_element_type=jnp.float32)
    acc[...] += jnp.concatenate([v1, v2], axis=0)

    @pl.when(pid == last)
    def _():
        r1 = jax.nn.relu(
            jnp.dot(rx[...], Wr1[...], preferred_element_type=jnp.float32)
            + br1[...])
        r = jax.nn.relu(
            jnp.dot(r1, Wr2[...], preferred_element_type=jnp.float32)
            + br2[...])
        d0 = dis0[0]
        v1f = acc[0:1, :]
        v2f = acc[1:2, :]
        out_ref[...] = (
            jnp.dot(r, W0[...] - W2[...], preferred_element_type=jnp.float32)
            + jnp.dot(-d0 * v1f, W1[...], preferred_element_type=jnp.float32)
            + jnp.dot(2.0 * d0 * v2f, W2[...],
                      preferred_element_type=jnp.float32)
            + bias[...])


def kernel(robot_x, human_x, edge_index, Wr1, br1, Wr2, br2, Wh1, bh1,
           Wh2, bh2, W0, W1, W2, bias):
    f32 = jnp.float32
    H = human_x.shape[1]
    n = H + 1
    E = edge_index.shape[1]
    npad = -(-n // _BLK) * _BLK
    grid = npad // _BLK

    src = edge_index[0]
    dst = edge_index[1]
    if E % _L:
        pad = _L - E % _L
        src = jnp.concatenate([src, jnp.full((pad,), npad - 1, jnp.int32)])
        dst = jnp.concatenate([dst, jnp.full((pad,), npad - 1, jnp.int32)])
    rows = src.shape[0] // _L
    src2 = src.reshape(rows, _L)
    dst2 = dst.reshape(rows, _L)

    deg_p, cun_p = _histo_kernel(rows, npad)(src2, dst2)

    hx = jnp.pad(human_x[0], ((1, npad - n), (0, 3)))
    rx = robot_x.reshape(1, 9)

    def full(a):
        return pl.BlockSpec(a.shape, lambda i: (0, 0))

    Wh1p = jnp.pad(Wh1, ((0, 3), (0, 0)))
    br1_2 = br1.reshape(1, -1)
    br2_2 = br2.reshape(1, -1)
    bh1_2 = bh1.reshape(1, -1)
    bh2_2 = bh2.reshape(1, -1)
    bias_2 = bias.reshape(1, -1)

    X, t1, cd, dis = pl.pallas_call(
        _tables_body,
        grid=(grid,),
        in_specs=[
            pl.BlockSpec((2, _BLK), lambda i: (0, i)),
            pl.BlockSpec((2, _BLK), lambda i: (0, i)),
            pl.BlockSpec((_BLK, 8), lambda i: (i, 0)),
            full(rx), full(Wh1p), full(bh1_2), full(Wh2), full(bh2_2),
            full(Wr1), full(br1_2), full(Wr2), full(br2_2),
        ],
        out_specs=[
            pl.BlockSpec((_BLK, 32), lambda i: (i, 0)),
            pl.BlockSpec((1, _BLK), lambda i: (0, i)),
            pl.BlockSpec((1, _BLK), lambda i: (0, i)),
            pl.BlockSpec((1, _BLK), lambda i: (0, i)),
        ],
        out_shape=[
            jax.ShapeDtypeStruct((npad, 32), f32),
            jax.ShapeDtypeStruct((1, npad), f32),
            jax.ShapeDtypeStruct((1, npad), f32),
            jax.ShapeDtypeStruct((1, npad), f32),
        ],
    )(deg_p, cun_p, hx, rx, Wh1p, bh1_2, Wh2, bh2_2, Wr1, br1_2, Wr2, br2_2)

    hp = _prop_kernel(rows, npad)(src2, dst2, cd.reshape(npad))

    out = pl.pallas_call(
        _final_body,
        grid=(grid,),
        in_specs=[
            pl.BlockSpec((_BLK, 32), lambda i: (i, 0)),
            pl.BlockSpec((1, _BLK), lambda i: (0, i)),
            pl.BlockSpec((1, _BLK), lambda i: (0, i)),
            pl.BlockSpec((2, _BLK), lambda i: (0, i)),
            full(rx), full(Wr1), full(br1_2), full(Wr2), full(br2_2),
            full(W0), full(W1), full(W2), full(bias_2),
        ],
        out_specs=pl.BlockSpec((1, 32), lambda i: (0, 0)),
        out_shape=jax.ShapeDtypeStruct((1, 32), f32),
        scratch_shapes=[
            pltpu.VMEM((2, 32), f32),
            pltpu.SMEM((1,), f32),
        ],
    )(X, t1, dis, hp, rx, Wr1, br1_2, Wr2, br2_2, W0, W1, W2, bias_2)

    return out


# speedup vs baseline: 112.3841x; 1.8914x over previous
import functools

import jax
import jax.numpy as jnp
from jax import lax
from jax.experimental import pallas as pl
from jax.experimental.pallas import tpu as pltpu
from jax.experimental.pallas import tpu_sc as plsc

_L = 128
_SLAB = 16
_NW = 32
_BLK = 2048


def _wid():
    return lax.axis_index("s") * 2 + lax.axis_index("c")


def _row_range(rows):
    wid = _wid()
    base, ext = rows // _NW, rows % _NW
    nrows = base + jnp.where(wid < ext, 1, 0).astype(jnp.int32)
    row0 = wid * base + jnp.minimum(wid, ext)
    return row0, nrows


def _zero_spmem(zbuf, sps, seg):
    sid = lax.axis_index("s")
    for i in range(seg // 16):
        zbuf[pl.ds(i * 16, 16)] = jnp.zeros((16,), jnp.float32)
    for sp in sps:
        pltpu.sync_copy(zbuf, sp.at[pl.ds(sid * seg, seg)])


def _copy_out(zbuf, sp, out, seg):
    cid = lax.axis_index("c")
    sid = lax.axis_index("s")
    sl = pl.ds(sid * seg, seg)
    pltpu.sync_copy(sp.at[sl], zbuf)
    pltpu.sync_copy(zbuf, out.at[cid, sl])


def _scan_slabs(src_hbm, dst_hbm, srcb, dstb, semL, row0, nrows, process):
    nslab = nrows // _SLAB

    def start_load(k, b):
        r0 = row0 + k * _SLAB
        pltpu.async_copy(src_hbm.at[pl.ds(r0, _SLAB), :], srcb.at[b], semL[b])
        pltpu.async_copy(dst_hbm.at[pl.ds(r0, _SLAB), :], dstb.at[b], semL[b])

    def wait_load(b):
        pltpu.make_async_copy(
            src_hbm.at[pl.ds(0, _SLAB), :], srcb.at[b], semL[b]).wait()
        pltpu.make_async_copy(
            dst_hbm.at[pl.ds(0, _SLAB), :], dstb.at[b], semL[b]).wait()

    @pl.when(nslab > 0)
    def _():
        start_load(0, 0)

    def pair_body(s2, c):
        for b in range(2):
            k = s2 * 2 + b
            wait_load(b)

            @pl.when(k + 1 < nslab)
            def _(k=k, b=b):
                start_load(k + 1, 1 - b)

            process(b)
        return c

    lax.fori_loop(0, nslab // 2, pair_body, 0)

    @pl.when(nslab % 2 == 1)
    def _():
        wait_load(0)
        process(0)

    return nslab


def _histo_kernel(rows, npad):
    seg = npad // 16
    mesh = plsc.VectorSubcoreMesh(core_axis_name="c", subcore_axis_name="s")

    @functools.partial(
        pl.kernel, mesh=mesh,
        compiler_params=pltpu.CompilerParams(use_tc_tiling_on_sc=False, needs_layout_passes=False),
        out_type=[jax.ShapeDtypeStruct((2, npad), jnp.float32),
                  jax.ShapeDtypeStruct((2, npad), jnp.float32)],
        scratch_types=[
            pltpu.VMEM((2, _SLAB, _L), jnp.int32),
            pltpu.VMEM((2, _SLAB, _L), jnp.int32),
            pltpu.VMEM((_L,), jnp.float32),
            pltpu.VMEM((_L,), jnp.float32),
            pltpu.VMEM((seg,), jnp.float32),
            pltpu.VMEM_SHARED((npad,), jnp.float32),
            pltpu.VMEM_SHARED((npad,), jnp.float32),
            pltpu.SemaphoreType.DMA,
            pltpu.SemaphoreType.DMA,
            pltpu.SemaphoreType.DMA,
        ],
    )
    def k1(src_hbm, dst_hbm, deg_out, cun_out,
           srcb, dstb, valb, ones, zbuf, deg_sp, cun_sp, semL0, semL1, semS):
        _zero_spmem(zbuf, (deg_sp, cun_sp), seg)
        for i in range(_L // 16):
            ones[pl.ds(i * 16, 16)] = jnp.ones((16,), jnp.float32)
        plsc.subcore_barrier()

        row0, nrows = _row_range(rows)

        def check_row(b, j):
            m = dstb[b, j, pl.ds(0, 16)] == 0
            for k in range(1, _L // 16):
                m = m | (dstb[b, j, pl.ds(k * 16, 16)] == 0)
            has0 = lax.reduce_or(m, axes=(0,))

            @pl.when(has0)
            def _():
                for k in range(_L // 16):
                    valb[pl.ds(k * 16, 16)] = jnp.where(
                        dstb[b, j, pl.ds(k * 16, 16)] == 0, 1.0, 0.0)
                pltpu.sync_copy(valb, cun_sp.at[srcb.at[b, j]], add=True)

        def process(b):
            for j in range(_SLAB):
                pltpu.async_copy(ones, deg_sp.at[srcb.at[b, j]], semS,
                                 add=True)
            for j in range(_SLAB):
                check_row(b, j)
            for j in range(_SLAB):
                pltpu.make_async_copy(ones, deg_sp.at[srcb.at[b, j]],
                                      semS).wait()

        nslab = _scan_slabs(src_hbm, dst_hbm, srcb, dstb, (semL0, semL1),
                            row0, nrows, process)

        def tail_body(t, c):
            r = row0 + nslab * _SLAB + t
            pltpu.sync_copy(src_hbm.at[r], srcb.at[0, 0])
            pltpu.sync_copy(dst_hbm.at[r], dstb.at[0, 0])
            pltpu.sync_copy(ones, deg_sp.at[srcb.at[0, 0]], add=True)
            check_row(0, 0)
            return c

        lax.fori_loop(0, nrows - nslab * _SLAB, tail_body, 0)

        plsc.subcore_barrier()
        _copy_out(zbuf, deg_sp, deg_out, seg)
        _copy_out(zbuf, cun_sp, cun_out, seg)

    return k1


def _prop_kernel(rows, npad):
    seg = npad // 16
    mesh = plsc.VectorSubcoreMesh(core_axis_name="c", subcore_axis_name="s")

    @functools.partial(
        pl.kernel, mesh=mesh,
        compiler_params=pltpu.CompilerParams(use_tc_tiling_on_sc=False, needs_layout_passes=False),
        out_type=jax.ShapeDtypeStruct((2, npad), jnp.float32),
        scratch_types=[
            pltpu.VMEM((2, _SLAB, _L), jnp.int32),
            pltpu.VMEM((2, _SLAB, _L), jnp.int32),
            pltpu.VMEM((_SLAB, _L), jnp.float32),
            pltpu.VMEM((seg,), jnp.float32),
            pltpu.VMEM_SHARED((npad,), jnp.float32),
            pltpu.SemaphoreType.DMA,
            pltpu.SemaphoreType.DMA,
            pltpu.SemaphoreType.DMA,
            pltpu.SemaphoreType.DMA,
        ],
    )
    def k3(src_hbm, dst_hbm, cd_hbm, hp_out, srcb, dstb, gbuf, zbuf, hp_sp,
           semL0, semL1, semG, semS):
        _zero_spmem(zbuf, (hp_sp,), seg)
        plsc.subcore_barrier()

        row0, nrows = _row_range(rows)

        def process(b):
            for j in range(_SLAB):
                pltpu.async_copy(cd_hbm.at[dstb.at[b, j]], gbuf.at[j], semG)
            for j in range(_SLAB):
                pltpu.make_async_copy(cd_hbm.at[dstb.at[b, j]], gbuf.at[j],
                                      semG).wait()
            for j in range(_SLAB):
                pltpu.async_copy(gbuf.at[j], hp_sp.at[srcb.at[b, j]], semS,
                                 add=True)
            for j in range(_SLAB):
                pltpu.make_async_copy(gbuf.at[j], hp_sp.at[srcb.at[b, j]],
                                      semS).wait()

        nslab = _scan_slabs(src_hbm, dst_hbm, srcb, dstb, (semL0, semL1),
                            row0, nrows, process)

        def tail_body(t, c):
            r = row0 + nslab * _SLAB + t
            pltpu.sync_copy(src_hbm.at[r], srcb.at[0, 0])
            pltpu.sync_copy(dst_hbm.at[r], dstb.at[0, 0])
            pltpu.sync_copy(cd_hbm.at[dstb.at[0, 0]], gbuf.at[0])
            pltpu.sync_copy(gbuf.at[0], hp_sp.at[srcb.at[0, 0]], add=True)
            return c

        lax.fori_loop(0, nrows - nslab * _SLAB, tail_body, 0)

        plsc.subcore_barrier()
        _copy_out(zbuf, hp_sp, hp_out, seg)

    return k3


def _tables_body(dp, cp, hx, rx, Wh1, bh1, Wh2, bh2, Wr1, br1, Wr2, br2,
                 X_ref, t1_ref, cd_ref, dis_ref):
    deg = dp[0:1, :] + dp[1:2, :]
    pos = deg > 0.0
    dis = jnp.where(pos, lax.rsqrt(jnp.maximum(deg, 1e-12)), 0.0)
    cun = cp[0:1, :] + cp[1:2, :]
    t1_ref[...] = cun * dis
    cd_ref[...] = jnp.where(pos, cun / jnp.maximum(deg, 1e-12), 0.0)
    dis_ref[...] = dis
    h1 = jax.nn.relu(
        jnp.dot(hx[...], Wh1[...], preferred_element_type=jnp.float32)
        + bh1[...])
    X_ref[...] = jax.nn.relu(
        jnp.dot(h1, Wh2[...], preferred_element_type=jnp.float32) + bh2[...])

    @pl.when(pl.program_id(0) == 0)
    def _():
        r1 = jax.nn.relu(
            jnp.dot(rx[...], Wr1[...], preferred_element_type=jnp.float32)
            + br1[...])
        r = jax.nn.relu(
            jnp.dot(r1, Wr2[...], preferred_element_type=jnp.float32)
            + br2[...])
        X_ref[0:1, :] = r


def _final_body(X, t1, dis, hp, rx, Wr1, br1, Wr2, br2, W0, W1, W2, bias,
                out_ref, acc, dis0):
    pid = pl.program_id(0)
    last = pl.num_programs(0) - 1

    @pl.when(pid == 0)
    def _():
        acc[...] = jnp.zeros_like(acc)
        dis0[0] = dis[0, 0]

    g = dis[...] * (hp[0:1, :] + hp[1:2, :])
    v1 = jnp.dot(t1[...], X[...], preferred_element_type=jnp.float32)
    v2 = jnp.dot(g, X[...], preferred_element_type=jnp.float32)
    acc[...] += jnp.concatenate([v1, v2], axis=0)

    @pl.when(pid == last)
    def _():
        r1 = jax.nn.relu(
            jnp.dot(rx[...], Wr1[...], preferred_element_type=jnp.float32)
            + br1[...])
        r = jax.nn.relu(
            jnp.dot(r1, Wr2[...], preferred_element_type=jnp.float32)
            + br2[...])
        d0 = dis0[0]
        v1f = acc[0:1, :]
        v2f = acc[1:2, :]
        out_ref[...] = (
            jnp.dot(r, W0[...] - W2[...], preferred_element_type=jnp.float32)
            + jnp.dot(-d0 * v1f, W1[...], preferred_element_type=jnp.float32)
            + jnp.dot(2.0 * d0 * v2f, W2[...],
                      preferred_element_type=jnp.float32)
            + bias[...])


def kernel(robot_x, human_x, edge_index, Wr1, br1, Wr2, br2, Wh1, bh1,
           Wh2, bh2, W0, W1, W2, bias):
    f32 = jnp.float32
    H = human_x.shape[1]
    n = H + 1
    E = edge_index.shape[1]
    npad = -(-n // _BLK) * _BLK
    grid = npad // _BLK

    src = edge_index[0]
    dst = edge_index[1]
    if E % _L:
        pad = _L - E % _L
        src = jnp.concatenate([src, jnp.full((pad,), npad - 1, jnp.int32)])
        dst = jnp.concatenate([dst, jnp.full((pad,), npad - 1, jnp.int32)])
    rows = src.shape[0] // _L
    src2 = src.reshape(rows, _L)
    dst2 = dst.reshape(rows, _L)

    deg_p, cun_p = _histo_kernel(rows, npad)(src2, dst2)

    hx = jnp.pad(human_x[0], ((1, npad - n), (0, 3)))
    rx = robot_x.reshape(1, 9)

    def full(a):
        return pl.BlockSpec(a.shape, lambda i: (0, 0))

    Wh1p = jnp.pad(Wh1, ((0, 3), (0, 0)))
    br1_2 = br1.reshape(1, -1)
    br2_2 = br2.reshape(1, -1)
    bh1_2 = bh1.reshape(1, -1)
    bh2_2 = bh2.reshape(1, -1)
    bias_2 = bias.reshape(1, -1)

    X, t1, cd, dis = pl.pallas_call(
        _tables_body,
        grid=(grid,),
        in_specs=[
            pl.BlockSpec((2, _BLK), lambda i: (0, i)),
            pl.BlockSpec((2, _BLK), lambda i: (0, i)),
            pl.BlockSpec((_BLK, 8), lambda i: (i, 0)),
            full(rx), full(Wh1p), full(bh1_2), full(Wh2), full(bh2_2),
            full(Wr1), full(br1_2), full(Wr2), full(br2_2),
        ],
        out_specs=[
            pl.BlockSpec((_BLK, 32), lambda i: (i, 0)),
            pl.BlockSpec((1, _BLK), lambda i: (0, i)),
            pl.BlockSpec((1, _BLK), lambda i: (0, i)),
            pl.BlockSpec((1, _BLK), lambda i: (0, i)),
        ],
        out_shape=[
            jax.ShapeDtypeStruct((npad, 32), f32),
            jax.ShapeDtypeStruct((1, npad), f32),
            jax.ShapeDtypeStruct((1, npad), f32),
            jax.ShapeDtypeStruct((1, npad), f32),
        ],
    )(deg_p, cun_p, hx, rx, Wh1p, bh1_2, Wh2, bh2_2, Wr1, br1_2, Wr2, br2_2)

    hp = _prop_kernel(rows, npad)(src2, dst2, cd.reshape(npad))

    out = pl.pallas_call(
        _final_body,
        grid=(grid,),
        in_specs=[
            pl.BlockSpec((_BLK, 32), lambda i: (i, 0)),
            pl.BlockSpec((1, _BLK), lambda i: (0, i)),
            pl.BlockSpec((1, _BLK), lambda i: (0, i)),
            pl.BlockSpec((2, _BLK), lambda i: (0, i)),
            full(rx), full(Wr1), full(br1_2), full(Wr2), full(br2_2),
            full(W0), full(W1), full(W2), full(bias_2),
        ],
        out_specs=pl.BlockSpec((1, 32), lambda i: (0, 0)),
        out_shape=jax.ShapeDtypeStruct((1, 32), f32),
        scratch_shapes=[
            pltpu.VMEM((2, 32), f32),
            pltpu.SMEM((1,), f32),
        ],
    )(X, t1, dis, hp, rx, Wr1, br1_2, Wr2, br2_2, W0, W1, W2, bias_2)

    return out


# speedup vs baseline: 152.6907x; 1.3586x over previous
import functools

import jax
import jax.numpy as jnp
from jax import lax
from jax.experimental import pallas as pl
from jax.experimental.pallas import tpu as pltpu
from jax.experimental.pallas import tpu_sc as plsc

_L = 128
_SLAB = 16
_NW = 32
_BLK = 2048


def _wid():
    return lax.axis_index("s") * 2 + lax.axis_index("c")


def _row_range(rows):
    wid = _wid()
    base, ext = rows // _NW, rows % _NW
    nrows = base + jnp.where(wid < ext, 1, 0).astype(jnp.int32)
    row0 = wid * base + jnp.minimum(wid, ext)
    return row0, nrows


def _zero_spmem(zbuf, sps, seg):
    sid = lax.axis_index("s")
    for i in range(seg // 16):
        zbuf[pl.ds(i * 16, 16)] = jnp.zeros((16,), jnp.float32)
    for sp in sps:
        pltpu.sync_copy(zbuf, sp.at[pl.ds(sid * seg, seg)])


def _copy_out(zbuf, sp, out, seg):
    cid = lax.axis_index("c")
    sid = lax.axis_index("s")
    sl = pl.ds(sid * seg, seg)
    pltpu.sync_copy(sp.at[sl], zbuf)
    pltpu.sync_copy(zbuf, out.at[cid, sl])


def _scan_slabs(src_hbm, dst_hbm, srcb, dstb, semL, row0, nrows, process):
    nslab = nrows // _SLAB

    def start_load(k, b):
        r0 = row0 + k * _SLAB
        pltpu.async_copy(src_hbm.at[pl.ds(r0, _SLAB), :], srcb.at[b], semL[b])
        pltpu.async_copy(dst_hbm.at[pl.ds(r0, _SLAB), :], dstb.at[b], semL[b])

    def wait_load(b):
        pltpu.make_async_copy(
            src_hbm.at[pl.ds(0, _SLAB), :], srcb.at[b], semL[b]).wait()
        pltpu.make_async_copy(
            dst_hbm.at[pl.ds(0, _SLAB), :], dstb.at[b], semL[b]).wait()

    @pl.when(nslab > 0)
    def _():
        start_load(0, 0)

    def pair_body(s2, c):
        for b in range(2):
            k = s2 * 2 + b
            wait_load(b)

            @pl.when(k + 1 < nslab)
            def _(k=k, b=b):
                start_load(k + 1, 1 - b)

            process(b)
        return c

    lax.fori_loop(0, nslab // 2, pair_body, 0)

    @pl.when(nslab % 2 == 1)
    def _():
        wait_load(0)
        process(0)

    return nslab


def _histo_kernel(rows, npad):
    seg = npad // 16
    mesh = plsc.VectorSubcoreMesh(core_axis_name="c", subcore_axis_name="s")

    @functools.partial(
        pl.kernel, mesh=mesh,
        compiler_params=pltpu.CompilerParams(use_tc_tiling_on_sc=False, needs_layout_passes=False),
        out_type=[jax.ShapeDtypeStruct((2, npad), jnp.float32),
                  jax.ShapeDtypeStruct((2, npad), jnp.float32)],
        scratch_types=[
            pltpu.VMEM((2, _SLAB, _L), jnp.int32),
            pltpu.VMEM((2, _SLAB, _L), jnp.int32),
            pltpu.VMEM((_L,), jnp.float32),
            pltpu.VMEM((_L,), jnp.float32),
            pltpu.VMEM((seg,), jnp.float32),
            pltpu.VMEM_SHARED((npad,), jnp.float32),
            pltpu.VMEM_SHARED((npad,), jnp.float32),
            pltpu.SemaphoreType.DMA,
            pltpu.SemaphoreType.DMA,
            pltpu.SemaphoreType.DMA,
        ],
    )
    def k1(src_hbm, dst_hbm, deg_out, cun_out,
           srcb, dstb, valb, ones, zbuf, deg_sp, cun_sp, semL0, semL1, semS):
        _zero_spmem(zbuf, (deg_sp, cun_sp), seg)
        for i in range(_L // 16):
            ones[pl.ds(i * 16, 16)] = jnp.ones((16,), jnp.float32)
        plsc.subcore_barrier()

        row0, nrows = _row_range(rows)

        def check_row(b, j):
            m = dstb[b, j, pl.ds(0, 16)] == 0
            for k in range(1, _L // 16):
                m = m | (dstb[b, j, pl.ds(k * 16, 16)] == 0)
            has0 = lax.reduce_or(m, axes=(0,))

            @pl.when(has0)
            def _():
                for k in range(_L // 16):
                    valb[pl.ds(k * 16, 16)] = jnp.where(
                        dstb[b, j, pl.ds(k * 16, 16)] == 0, 1.0, 0.0)
                pltpu.sync_copy(valb, cun_sp.at[srcb.at[b, j]], add=True)

        def process(b):
            for j in range(_SLAB):
                pltpu.async_copy(ones, deg_sp.at[srcb.at[b, j]], semS,
                                 add=True)
            for j in range(_SLAB):
                check_row(b, j)
            for j in range(_SLAB):
                pltpu.make_async_copy(ones, deg_sp.at[srcb.at[b, j]],
                                      semS).wait()

        nslab = _scan_slabs(src_hbm, dst_hbm, srcb, dstb, (semL0, semL1),
                            row0, nrows, process)

        def tail_body(t, c):
            r = row0 + nslab * _SLAB + t
            pltpu.sync_copy(src_hbm.at[r], srcb.at[0, 0])
            pltpu.sync_copy(dst_hbm.at[r], dstb.at[0, 0])
            pltpu.sync_copy(ones, deg_sp.at[srcb.at[0, 0]], add=True)
            check_row(0, 0)
            return c

        lax.fori_loop(0, nrows - nslab * _SLAB, tail_body, 0)

        plsc.subcore_barrier()
        _copy_out(zbuf, deg_sp, deg_out, seg)
        _copy_out(zbuf, cun_sp, cun_out, seg)

    return k1


def _prop_kernel(rows, npad):
    seg = npad // 16
    mesh = plsc.VectorSubcoreMesh(core_axis_name="c", subcore_axis_name="s")

    @functools.partial(
        pl.kernel, mesh=mesh,
        compiler_params=pltpu.CompilerParams(use_tc_tiling_on_sc=False, needs_layout_passes=False),
        out_type=jax.ShapeDtypeStruct((2, npad), jnp.float32),
        scratch_types=[
            pltpu.VMEM((2, _SLAB, _L), jnp.int32),
            pltpu.VMEM((2, _SLAB, _L), jnp.int32),
            pltpu.VMEM((_SLAB, _L), jnp.float32),
            pltpu.VMEM((seg,), jnp.float32),
            pltpu.VMEM((seg,), jnp.float32),
            pltpu.VMEM((seg,), jnp.float32),
            pltpu.VMEM_SHARED((npad,), jnp.float32),
            pltpu.VMEM_SHARED((npad,), jnp.float32),
            pltpu.SemaphoreType.DMA,
            pltpu.SemaphoreType.DMA,
            pltpu.SemaphoreType.DMA,
            pltpu.SemaphoreType.DMA,
        ],
    )
    def k2(src_hbm, dst_hbm, deg_hbm, cun_hbm, hp_out,
           srcb, dstb, gbuf, zbuf, tb0, tb1, cd_sp, hp_sp,
           semL0, semL1, semG, semS):
        sid = lax.axis_index("s")
        _zero_spmem(zbuf, (hp_sp,), seg)

        sl = pl.ds(sid * seg, seg)
        pltpu.sync_copy(deg_hbm.at[0, sl], tb0)
        pltpu.sync_copy(deg_hbm.at[1, sl], tb1)
        for i in range(seg // 16):
            ds = pl.ds(i * 16, 16)
            tb0[ds] = tb0[ds] + tb1[ds]
        pltpu.sync_copy(cun_hbm.at[0, sl], tb1)
        pltpu.sync_copy(cun_hbm.at[1, sl], zbuf)
        for i in range(seg // 16):
            ds = pl.ds(i * 16, 16)
            d = tb0[ds]
            cn = tb1[ds] + zbuf[ds]
            tb0[ds] = jnp.where(d > 0.0, cn / jnp.maximum(d, 1.0), 0.0)
        pltpu.sync_copy(tb0, cd_sp.at[sl])
        for i in range(seg // 16):
            zbuf[pl.ds(i * 16, 16)] = jnp.zeros((16,), jnp.float32)
        plsc.subcore_barrier()

        row0, nrows = _row_range(rows)

        def process(b):
            for j in range(_SLAB):
                pltpu.async_copy(cd_sp.at[dstb.at[b, j]], gbuf.at[j], semG)
            for j in range(_SLAB):
                pltpu.make_async_copy(cd_sp.at[dstb.at[b, j]], gbuf.at[j],
                                      semG).wait()
            for j in range(_SLAB):
                pltpu.async_copy(gbuf.at[j], hp_sp.at[srcb.at[b, j]], semS,
                                 add=True)
            for j in range(_SLAB):
                pltpu.make_async_copy(gbuf.at[j], hp_sp.at[srcb.at[b, j]],
                                      semS).wait()

        nslab = _scan_slabs(src_hbm, dst_hbm, srcb, dstb, (semL0, semL1),
                            row0, nrows, process)

        def tail_body(t, c):
            r = row0 + nslab * _SLAB + t
            pltpu.sync_copy(src_hbm.at[r], srcb.at[0, 0])
            pltpu.sync_copy(dst_hbm.at[r], dstb.at[0, 0])
            pltpu.sync_copy(cd_sp.at[dstb.at[0, 0]], gbuf.at[0])
            pltpu.sync_copy(gbuf.at[0], hp_sp.at[srcb.at[0, 0]], add=True)
            return c

        lax.fori_loop(0, nrows - nslab * _SLAB, tail_body, 0)

        plsc.subcore_barrier()
        _copy_out(zbuf, hp_sp, hp_out, seg)

    return k2


def _final_body(dp, cp, hp, hx, rx, Wh1, bh1, Wh2, bh2,
                Wr1, br1, Wr2, br2, W0, W1, W2, bias,
                out_ref, acc, dis0):
    pid = pl.program_id(0)
    last = pl.num_programs(0) - 1

    deg = dp[0:1, :] + dp[1:2, :]
    pos = deg > 0.0
    dis = jnp.where(pos, lax.rsqrt(jnp.maximum(deg, 1e-12)), 0.0)

    @pl.when(pid == 0)
    def _():
        acc[...] = jnp.zeros_like(acc)
        dis0[0] = dis[0, 0]

    t1 = (cp[0:1, :] + cp[1:2, :]) * dis
    g = dis * (hp[0:1, :] + hp[1:2, :])

    h1 = jax.nn.relu(
        jnp.dot(hx[...], Wh1[...], preferred_element_type=jnp.float32)
        + bh1[...])
    X = jax.nn.relu(
        jnp.dot(h1, Wh2[...], preferred_element_type=jnp.float32) + bh2[...])

    r1 = jax.nn.relu(
        jnp.dot(rx[...], Wr1[...], preferred_element_type=jnp.float32)
        + br1[...])
    r = jax.nn.relu(
        jnp.dot(r1, Wr2[...], preferred_element_type=jnp.float32)
        + br2[...])

    @pl.when(pid == 0)
    def _():
        X_ref0 = jnp.where(
            lax.broadcasted_iota(jnp.int32, (X.shape[0], 1), 0) == 0, r, X)
        acc[...] += jnp.concatenate(
            [jnp.dot(t1, X_ref0, preferred_element_type=jnp.float32),
             jnp.dot(g, X_ref0, preferred_element_type=jnp.float32)], axis=0)

    @pl.when(pid != 0)
    def _():
        acc[...] += jnp.concatenate(
            [jnp.dot(t1, X, preferred_element_type=jnp.float32),
             jnp.dot(g, X, preferred_element_type=jnp.float32)], axis=0)

    @pl.when(pid == last)
    def _():
        d0 = dis0[0]
        v1f = acc[0:1, :]
        v2f = acc[1:2, :]
        out_ref[...] = (
            jnp.dot(r, W0[...] - W2[...], preferred_element_type=jnp.float32)
            + jnp.dot(-d0 * v1f, W1[...], preferred_element_type=jnp.float32)
            + jnp.dot(2.0 * d0 * v2f, W2[...],
                      preferred_element_type=jnp.float32)
            + bias[...])


def kernel(robot_x, human_x, edge_index, Wr1, br1, Wr2, br2, Wh1, bh1,
           Wh2, bh2, W0, W1, W2, bias):
    f32 = jnp.float32
    H = human_x.shape[1]
    n = H + 1
    E = edge_index.shape[1]
    npad = -(-n // _BLK) * _BLK
    grid = npad // _BLK

    src = edge_index[0]
    dst = edge_index[1]
    if E % _L:
        pad = _L - E % _L
        src = jnp.concatenate([src, jnp.full((pad,), npad - 1, jnp.int32)])
        dst = jnp.concatenate([dst, jnp.full((pad,), npad - 1, jnp.int32)])
    rows = src.shape[0] // _L
    src2 = src.reshape(rows, _L)
    dst2 = dst.reshape(rows, _L)

    deg_p, cun_p = _histo_kernel(rows, npad)(src2, dst2)
    hp = _prop_kernel(rows, npad)(src2, dst2, deg_p, cun_p)

    hx = jnp.pad(human_x[0], ((1, npad - n), (0, 3)))
    rx = robot_x.reshape(1, 9)
    Wh1p = jnp.pad(Wh1, ((0, 3), (0, 0)))
    br1_2 = br1.reshape(1, -1)
    br2_2 = br2.reshape(1, -1)
    bh1_2 = bh1.reshape(1, -1)
    bh2_2 = bh2.reshape(1, -1)
    bias_2 = bias.reshape(1, -1)

    def full(a):
        return pl.BlockSpec(a.shape, lambda i: (0, 0))

    out = pl.pallas_call(
        _final_body,
        grid=(grid,),
        in_specs=[
            pl.BlockSpec((2, _BLK), lambda i: (0, i)),
            pl.BlockSpec((2, _BLK), lambda i: (0, i)),
            pl.BlockSpec((2, _BLK), lambda i: (0, i)),
            pl.BlockSpec((_BLK, 8), lambda i: (i, 0)),
            full(rx), full(Wh1p), full(bh1_2), full(Wh2), full(bh2_2),
            full(Wr1), full(br1_2), full(Wr2), full(br2_2),
            full(W0), full(W1), full(W2), full(bias_2),
        ],
        out_specs=pl.BlockSpec((1, 32), lambda i: (0, 0)),
        out_shape=jax.ShapeDtypeStruct((1, 32), f32),
        scratch_shapes=[
            pltpu.VMEM((2, 32), f32),
            pltpu.SMEM((1,), f32),
        ],
    )(deg_p, cun_p, hp, hx, rx, Wh1p, bh1_2, Wh2, bh2_2,
      Wr1, br1_2, Wr2, br2_2, W0, W1, W2, bias_2)

    return out


# speedup vs baseline: 176.2163x; 1.1541x over previous
import functools

import jax
import jax.numpy as jnp
from jax import lax
from jax.experimental import pallas as pl
from jax.experimental.pallas import tpu as pltpu
from jax.experimental.pallas import tpu_sc as plsc

_L = 128
_SLAB = 16
_NW = 32
_BLK = 8192


def _wid():
    return lax.axis_index("s") * 2 + lax.axis_index("c")


def _row_range(rows):
    wid = _wid()
    base, ext = rows // _NW, rows % _NW
    nrows = base + jnp.where(wid < ext, 1, 0).astype(jnp.int32)
    row0 = wid * base + jnp.minimum(wid, ext)
    return row0, nrows


def _zero_spmem(zbuf, sps, seg):
    sid = lax.axis_index("s")
    for i in range(seg // 16):
        zbuf[pl.ds(i * 16, 16)] = jnp.zeros((16,), jnp.float32)
    for sp in sps:
        pltpu.sync_copy(zbuf, sp.at[pl.ds(sid * seg, seg)])


def _copy_out(zbuf, sp, out, seg):
    cid = lax.axis_index("c")
    sid = lax.axis_index("s")
    sl = pl.ds(sid * seg, seg)
    pltpu.sync_copy(sp.at[sl], zbuf)
    pltpu.sync_copy(zbuf, out.at[cid, sl])


def _scan_slabs(src_hbm, dst_hbm, srcb, dstb, semL, row0, nrows, process):
    nslab = nrows // _SLAB

    def start_load(k, b):
        r0 = row0 + k * _SLAB
        pltpu.async_copy(src_hbm.at[pl.ds(r0, _SLAB), :], srcb.at[b], semL[b])
        pltpu.async_copy(dst_hbm.at[pl.ds(r0, _SLAB), :], dstb.at[b], semL[b])

    def wait_load(b):
        pltpu.make_async_copy(
            src_hbm.at[pl.ds(0, _SLAB), :], srcb.at[b], semL[b]).wait()
        pltpu.make_async_copy(
            dst_hbm.at[pl.ds(0, _SLAB), :], dstb.at[b], semL[b]).wait()

    @pl.when(nslab > 0)
    def _():
        start_load(0, 0)

    def pair_body(s2, c):
        for b in range(2):
            k = s2 * 2 + b
            wait_load(b)

            @pl.when(k + 1 < nslab)
            def _(k=k, b=b):
                start_load(k + 1, 1 - b)

            process(b, k)
        return c

    lax.fori_loop(0, nslab // 2, pair_body, 0)

    @pl.when(nslab % 2 == 1)
    def _():
        wait_load(0)
        process(0, nslab - 1)

    return nslab


def _drain_slab_bytes(src_hbm, proxy, sem):
    pltpu.make_async_copy(src_hbm.at[pl.ds(0, _SLAB), :], proxy, sem).wait()


def _histo_kernel(rows, npad):
    seg = npad // 16
    mesh = plsc.VectorSubcoreMesh(core_axis_name="c", subcore_axis_name="s")

    @functools.partial(
        pl.kernel, mesh=mesh,
        compiler_params=pltpu.CompilerParams(use_tc_tiling_on_sc=False, needs_layout_passes=False),
        out_type=[jax.ShapeDtypeStruct((2, npad), jnp.float32),
                  jax.ShapeDtypeStruct((2, npad), jnp.float32)],
        scratch_types=[
            pltpu.VMEM((2, _SLAB, _L), jnp.int32),
            pltpu.VMEM((2, _SLAB, _L), jnp.int32),
            pltpu.VMEM((_L,), jnp.float32),
            pltpu.VMEM((_L,), jnp.float32),
            pltpu.VMEM((seg,), jnp.float32),
            pltpu.VMEM_SHARED((npad,), jnp.float32),
            pltpu.VMEM_SHARED((npad,), jnp.float32),
            pltpu.SemaphoreType.DMA,
            pltpu.SemaphoreType.DMA,
            pltpu.SemaphoreType.DMA,
        ],
    )
    def k1(src_hbm, dst_hbm, deg_out, cun_out,
           srcb, dstb, valb, ones, zbuf, deg_sp, cun_sp, semL0, semL1, semS):
        _zero_spmem(zbuf, (deg_sp, cun_sp), seg)
        for i in range(_L // 16):
            ones[pl.ds(i * 16, 16)] = jnp.ones((16,), jnp.float32)
        plsc.subcore_barrier()

        row0, nrows = _row_range(rows)

        def check_row(b, j):
            m = dstb[b, j, pl.ds(0, 16)] == 0
            for k in range(1, _L // 16):
                m = m | (dstb[b, j, pl.ds(k * 16, 16)] == 0)
            has0 = lax.reduce_or(m, axes=(0,))

            @pl.when(has0)
            def _():
                for k in range(_L // 16):
                    valb[pl.ds(k * 16, 16)] = jnp.where(
                        dstb[b, j, pl.ds(k * 16, 16)] == 0, 1.0, 0.0)
                pltpu.sync_copy(valb, cun_sp.at[srcb.at[b, j]], add=True)

        def process(b, k):
            del k
            for j in range(_SLAB):
                pltpu.async_copy(ones, deg_sp.at[srcb.at[b, j]], semS,
                                 add=True)
            for j in range(_SLAB):
                check_row(b, j)

        nslab = _scan_slabs(src_hbm, dst_hbm, srcb, dstb, (semL0, semL1),
                            row0, nrows, process)

        def tail_body(t, c):
            r = row0 + nslab * _SLAB + t
            pltpu.sync_copy(src_hbm.at[r], srcb.at[0, 0])
            pltpu.sync_copy(dst_hbm.at[r], dstb.at[0, 0])
            pltpu.sync_copy(ones, deg_sp.at[srcb.at[0, 0]], add=True)
            check_row(0, 0)
            return c

        lax.fori_loop(0, nrows - nslab * _SLAB, tail_body, 0)

        def drain_body(s, c):
            _drain_slab_bytes(src_hbm, srcb.at[0], semS)
            return c

        lax.fori_loop(0, nslab, drain_body, 0)

        plsc.subcore_barrier()
        _copy_out(zbuf, deg_sp, deg_out, seg)
        _copy_out(zbuf, cun_sp, cun_out, seg)

    return k1


def _prop_kernel(rows, npad):
    seg = npad // 16
    mesh = plsc.VectorSubcoreMesh(core_axis_name="c", subcore_axis_name="s")

    @functools.partial(
        pl.kernel, mesh=mesh,
        compiler_params=pltpu.CompilerParams(use_tc_tiling_on_sc=False, needs_layout_passes=False),
        out_type=jax.ShapeDtypeStruct((2, npad), jnp.float32),
        scratch_types=[
            pltpu.VMEM((2, _SLAB, _L), jnp.int32),
            pltpu.VMEM((2, _SLAB, _L), jnp.int32),
            pltpu.VMEM((2, _SLAB, _L), jnp.float32),
            pltpu.VMEM((seg,), jnp.float32),
            pltpu.VMEM((seg,), jnp.float32),
            pltpu.VMEM((seg,), jnp.float32),
            pltpu.VMEM_SHARED((npad,), jnp.float32),
            pltpu.VMEM_SHARED((npad,), jnp.float32),
            pltpu.SemaphoreType.DMA,
            pltpu.SemaphoreType.DMA,
            pltpu.SemaphoreType.DMA,
            pltpu.SemaphoreType.DMA,
            pltpu.SemaphoreType.DMA,
        ],
    )
    def k2(src_hbm, dst_hbm, deg_hbm, cun_hbm, hp_out,
           srcb, dstb, gbuf, zbuf, tb0, tb1, cd_sp, hp_sp,
           semL0, semL1, semG, semS0, semS1):
        sid = lax.axis_index("s")
        _zero_spmem(zbuf, (hp_sp,), seg)

        sl = pl.ds(sid * seg, seg)
        pltpu.sync_copy(deg_hbm.at[0, sl], tb0)
        pltpu.sync_copy(deg_hbm.at[1, sl], tb1)
        for i in range(seg // 16):
            ds = pl.ds(i * 16, 16)
            tb0[ds] = tb0[ds] + tb1[ds]
        pltpu.sync_copy(cun_hbm.at[0, sl], tb1)
        pltpu.sync_copy(cun_hbm.at[1, sl], zbuf)
        for i in range(seg // 16):
            ds = pl.ds(i * 16, 16)
            d = tb0[ds]
            cn = tb1[ds] + zbuf[ds]
            tb0[ds] = jnp.where(d > 0.0, cn / jnp.maximum(d, 1.0), 0.0)
        pltpu.sync_copy(tb0, cd_sp.at[sl])
        for i in range(seg // 16):
            zbuf[pl.ds(i * 16, 16)] = jnp.zeros((16,), jnp.float32)
        plsc.subcore_barrier()

        row0, nrows = _row_range(rows)
        semS = (semS0, semS1)

        def process(b, k):
            @pl.when(k >= 2)
            def _():
                _drain_slab_bytes(src_hbm, srcb.at[b], semS[b])

            for j in range(_SLAB):
                pltpu.async_copy(cd_sp.at[dstb.at[b, j]], gbuf.at[b, j],
                                 semG)
            _drain_slab_bytes(src_hbm, srcb.at[b], semG)
            for j in range(_SLAB):
                pltpu.async_copy(gbuf.at[b, j], hp_sp.at[srcb.at[b, j]],
                                 semS[b], add=True)

        nslab = _scan_slabs(src_hbm, dst_hbm, srcb, dstb, (semL0, semL1),
                            row0, nrows, process)

        @pl.when(nslab >= 1)
        def _():
            _drain_slab_bytes(src_hbm, srcb.at[0], semS0)

        @pl.when(nslab >= 2)
        def _():
            _drain_slab_bytes(src_hbm, srcb.at[1], semS1)

        def tail_body(t, c):
            r = row0 + nslab * _SLAB + t
            pltpu.sync_copy(src_hbm.at[r], srcb.at[0, 0])
            pltpu.sync_copy(dst_hbm.at[r], dstb.at[0, 0])
            pltpu.sync_copy(cd_sp.at[dstb.at[0, 0]], gbuf.at[0, 0])
            pltpu.sync_copy(gbuf.at[0, 0], hp_sp.at[srcb.at[0, 0]], add=True)
            return c

        lax.fori_loop(0, nrows - nslab * _SLAB, tail_body, 0)

        plsc.subcore_barrier()
        _copy_out(zbuf, hp_sp, hp_out, seg)

    return k2


def _final_body(dp, cp, hp, hx, rx, Wh1, bh1, Wh2, bh2,
                Wr1, br1, Wr2, br2, W0, W1, W2, bias,
                out_ref, acc, dis0):
    pid = pl.program_id(0)
    last = pl.num_programs(0) - 1

    deg = dp[0:1, :] + dp[1:2, :]
    pos = deg > 0.0
    dis = jnp.where(pos, lax.rsqrt(jnp.maximum(deg, 1e-12)), 0.0)

    @pl.when(pid == 0)
    def _():
        acc[...] = jnp.zeros_like(acc)
        dis0[0] = dis[0, 0]

    t1 = (cp[0:1, :] + cp[1:2, :]) * dis
    g = dis * (hp[0:1, :] + hp[1:2, :])

    h1 = jax.nn.relu(
        jnp.dot(hx[...], Wh1[...], preferred_element_type=jnp.float32)
        + bh1[...])
    X = jax.nn.relu(
        jnp.dot(h1, Wh2[...], preferred_element_type=jnp.float32) + bh2[...])

    r1 = jax.nn.relu(
        jnp.dot(rx[...], Wr1[...], preferred_element_type=jnp.float32)
        + br1[...])
    r = jax.nn.relu(
        jnp.dot(r1, Wr2[...], preferred_element_type=jnp.float32)
        + br2[...])

    @pl.when(pid == 0)
    def _():
        X_ref0 = jnp.where(
            lax.broadcasted_iota(jnp.int32, (X.shape[0], 1), 0) == 0, r, X)
        acc[...] += jnp.concatenate(
            [jnp.dot(t1, X_ref0, preferred_element_type=jnp.float32),
             jnp.dot(g, X_ref0, preferred_element_type=jnp.float32)], axis=0)

    @pl.when(pid != 0)
    def _():
        acc[...] += jnp.concatenate(
            [jnp.dot(t1, X, preferred_element_type=jnp.float32),
             jnp.dot(g, X, preferred_element_type=jnp.float32)], axis=0)

    @pl.when(pid == last)
    def _():
        d0 = dis0[0]
        v1f = acc[0:1, :]
        v2f = acc[1:2, :]
        out_ref[...] = (
            jnp.dot(r, W0[...] - W2[...], preferred_element_type=jnp.float32)
            + jnp.dot(-d0 * v1f, W1[...], preferred_element_type=jnp.float32)
            + jnp.dot(2.0 * d0 * v2f, W2[...],
                      preferred_element_type=jnp.float32)
            + bias[...])


def kernel(robot_x, human_x, edge_index, Wr1, br1, Wr2, br2, Wh1, bh1,
           Wh2, bh2, W0, W1, W2, bias):
    f32 = jnp.float32
    H = human_x.shape[1]
    n = H + 1
    E = edge_index.shape[1]
    npad = -(-n // _BLK) * _BLK
    grid = npad // _BLK

    src = edge_index[0]
    dst = edge_index[1]
    if E % _L:
        pad = _L - E % _L
        src = jnp.concatenate([src, jnp.full((pad,), npad - 1, jnp.int32)])
        dst = jnp.concatenate([dst, jnp.full((pad,), npad - 1, jnp.int32)])
    rows = src.shape[0] // _L
    src2 = src.reshape(rows, _L)
    dst2 = dst.reshape(rows, _L)

    deg_p, cun_p = _histo_kernel(rows, npad)(src2, dst2)
    hp = _prop_kernel(rows, npad)(src2, dst2, deg_p, cun_p)

    hx = jnp.pad(human_x[0], ((1, npad - n), (0, 3)))
    rx = robot_x.reshape(1, 9)
    Wh1p = jnp.pad(Wh1, ((0, 3), (0, 0)))
    br1_2 = br1.reshape(1, -1)
    br2_2 = br2.reshape(1, -1)
    bh1_2 = bh1.reshape(1, -1)
    bh2_2 = bh2.reshape(1, -1)
    bias_2 = bias.reshape(1, -1)

    def full(a):
        return pl.BlockSpec(a.shape, lambda i: (0, 0))

    out = pl.pallas_call(
        _final_body,
        grid=(grid,),
        in_specs=[
            pl.BlockSpec((2, _BLK), lambda i: (0, i)),
            pl.BlockSpec((2, _BLK), lambda i: (0, i)),
            pl.BlockSpec((2, _BLK), lambda i: (0, i)),
            pl.BlockSpec((_BLK, 8), lambda i: (i, 0)),
            full(rx), full(Wh1p), full(bh1_2), full(Wh2), full(bh2_2),
            full(Wr1), full(br1_2), full(Wr2), full(br2_2),
            full(W0), full(W1), full(W2), full(bias_2),
        ],
        out_specs=pl.BlockSpec((1, 32), lambda i: (0, 0)),
        out_shape=jax.ShapeDtypeStruct((1, 32), f32),
        scratch_shapes=[
            pltpu.VMEM((2, 32), f32),
            pltpu.SMEM((1,), f32),
        ],
    )(deg_p, cun_p, hp, hx, rx, Wh1p, bh1_2, Wh2, bh2_2,
      Wr1, br1_2, Wr2, br2_2, W0, W1, W2, bias_2)

    return out
